# baseline (speedup 1.0000x reference)
import functools

import jax
import jax.numpy as jnp
from jax import lax
from jax.experimental import pallas as pl
from jax.experimental.pallas import tpu as pltpu
from jax.experimental.pallas import tpu_sc as plsc

_N = 100000
_BATCH = 16384
_NUM_CORES = 2
_NUM_SUBCORES = 16
_NW = _NUM_CORES * _NUM_SUBCORES
_BPW = _BATCH // _NW
_CHUNK = 128
_W = 8
_CPT = 6256
_CPT_LAST = _N - 15 * _CPT
_NCH = 392
_PACK_PAD = 6272
_L = 16

_mesh = plsc.VectorSubcoreMesh(
    core_axis_name="c",
    subcore_axis_name="s",
    num_cores=_NUM_CORES,
    num_subcores=_NUM_SUBCORES,
)
_params = pltpu.CompilerParams(
    use_tc_tiling_on_sc=False, needs_layout_passes=False)


def _pack_block(rott, trt, ppt, packed_out, colstack, packed_v, sem,
                cam0, ncams):
    copies = []
    for j in range(3):
        copies.append(pltpu.async_copy(
            rott.at[j, pl.ds(cam0, ncams)], colstack.at[j, pl.ds(0, ncams)],
            sem))
        copies.append(pltpu.async_copy(
            trt.at[j, pl.ds(cam0, ncams)],
            colstack.at[3 + j, pl.ds(0, ncams)], sem))
    for j in range(2):
        copies.append(pltpu.async_copy(
            ppt.at[j, pl.ds(cam0, ncams)],
            colstack.at[6 + j, pl.ds(0, ncams)], sem))
    for c in copies:
        c.wait()

    lane = lax.iota(jnp.int32, _L)
    cols = [jnp.full((_L,), j, jnp.int32) for j in range(_W)]

    def body(c, carry):
        for k in range(8):
            ch = c * 8 + k
            rows = lane + ch * _L
            for j in range(_W):
                v = colstack[j, pl.ds(ch * _L, _L)]
                plsc.store_scatter(packed_v, [rows, cols[j]], v)
        return carry

    lax.fori_loop(0, _NCH // 8, body, 0)
    pltpu.sync_copy(
        packed_v.at[pl.ds(0, ncams)],
        packed_out.at[pl.ds(cam0, ncams)])


def _body(rott, trt, ppt, fv, idx_hbm,
          packed_out, rvt_out, trt_out, fg_out, ppt_out,
          colstack, packed_v, idx_v, rows_v, fbuf, colbuf, sem, fsem):
    cid = lax.axis_index("c")
    sid = lax.axis_index("s")

    @pl.when(sid < _NUM_SUBCORES - 1)
    def _():
        _pack_block(rott, trt, ppt, packed_out, colstack, packed_v, sem,
                    sid * _CPT, _CPT)

    @pl.when(sid == _NUM_SUBCORES - 1)
    def _():
        _pack_block(rott, trt, ppt, packed_out, colstack, packed_v, sem,
                    (_NUM_SUBCORES - 1) * _CPT, _CPT_LAST)

    plsc.subcore_barrier()

    wid = sid * _NUM_CORES + cid
    base = wid * _BPW
    pltpu.sync_copy(idx_hbm.at[pl.ds(base, _BPW)], idx_v)
    copies = []
    for j in range(_BPW // _CHUNK):
        d = pl.ds(j * _CHUNK, _CHUNK)
        copies.append(pltpu.async_copy(
            packed_out.at[idx_v.at[d]], rows_v.at[d], sem))
        copies.append(pltpu.async_copy(fv.at[idx_v.at[d]], fbuf.at[d], fsem))
    for c in copies:
        c.wait()

    lane = lax.iota(jnp.int32, _L)
    for j in range(_W):
        jv = jnp.full((_L,), j, jnp.int32)
        for c in range(_BPW // _L):
            v = plsc.load_gather(rows_v, [c * _L + lane, jv])
            colbuf[j, pl.ds(c * _L, _L)] = v

    for j in range(3):
        pltpu.sync_copy(colbuf.at[j],
                        rvt_out.at[pl.ds(j * _BATCH + base, _BPW)])
        pltpu.sync_copy(colbuf.at[3 + j],
                        trt_out.at[pl.ds(j * _BATCH + base, _BPW)])
    for j in range(2):
        pltpu.sync_copy(colbuf.at[6 + j],
                        ppt_out.at[pl.ds(j * _BATCH + base, _BPW)])
    pltpu.sync_copy(fbuf, fg_out.at[pl.ds(base, _BPW)])


_fused = functools.partial(
    pl.kernel,
    mesh=_mesh,
    compiler_params=_params,
    out_type=(
        jax.ShapeDtypeStruct((_N, _W), jnp.float32),
        jax.ShapeDtypeStruct((3 * _BATCH,), jnp.float32),
        jax.ShapeDtypeStruct((3 * _BATCH,), jnp.float32),
        jax.ShapeDtypeStruct((_BATCH,), jnp.float32),
        jax.ShapeDtypeStruct((2 * _BATCH,), jnp.float32),
    ),
    scratch_types=[
        pltpu.VMEM((_W, _PACK_PAD), jnp.float32),
        pltpu.VMEM((_PACK_PAD, _W), jnp.float32),
        pltpu.VMEM((_BPW,), jnp.int32),
        pltpu.VMEM((_BPW, _W), jnp.float32),
        pltpu.VMEM((_BPW,), jnp.float32),
        pltpu.VMEM((_W, _BPW), jnp.float32),
        pltpu.SemaphoreType.DMA,
        pltpu.SemaphoreType.DMA,
    ],
)(_body)


def kernel(rotvecs, translations, f, pp, camera_idxs):
    idx = camera_idxs.astype(jnp.int32)
    _, rvt, trt, fg, ppt = _fused(rotvecs.T, translations.T, pp.T, f, idx)
    return (rvt.reshape(3, _BATCH).T, trt.reshape(3, _BATCH).T, fg,
            ppt.reshape(2, _BATCH).T)


# speedup vs baseline: 1.0606x; 1.0606x over previous
import functools

import jax
import jax.numpy as jnp
from jax import lax
from jax.experimental import pallas as pl
from jax.experimental.pallas import tpu as pltpu
from jax.experimental.pallas import tpu_sc as plsc

_N = 100000
_BATCH = 16384
_NUM_CORES = 2
_NUM_SUBCORES = 16
_NW = _NUM_CORES * _NUM_SUBCORES
_BPW = _BATCH // _NW
_CHUNK = 128
_W = 8
_CPT = 3128
_CPT_LAST = _N - 31 * _CPT
_NCH = 200
_NCH_LAST = 192
_PACK_PAD = 3200
_L = 16

_mesh = plsc.VectorSubcoreMesh(
    core_axis_name="c",
    subcore_axis_name="s",
    num_cores=_NUM_CORES,
    num_subcores=_NUM_SUBCORES,
)
_params = pltpu.CompilerParams(
    use_tc_tiling_on_sc=False, needs_layout_passes=False)


def _pack_block(rott, trt, ppt, packed_out, colstack, packed_v, sem,
                cam0, ncams, nch):
    copies = []
    for j in range(3):
        copies.append(pltpu.async_copy(
            rott.at[j, pl.ds(cam0, ncams)], colstack.at[j, pl.ds(0, ncams)],
            sem))
        copies.append(pltpu.async_copy(
            trt.at[j, pl.ds(cam0, ncams)],
            colstack.at[3 + j, pl.ds(0, ncams)], sem))
    for j in range(2):
        copies.append(pltpu.async_copy(
            ppt.at[j, pl.ds(cam0, ncams)],
            colstack.at[6 + j, pl.ds(0, ncams)], sem))
    for c in copies:
        c.wait()

    lane = lax.iota(jnp.int32, _L)
    cols = [jnp.full((_L,), j, jnp.int32) for j in range(_W)]

    def body(c, carry):
        for k in range(8):
            ch = c * 8 + k
            rows = lane + ch * _L
            for j in range(_W):
                v = colstack[j, pl.ds(ch * _L, _L)]
                plsc.store_scatter(packed_v, [rows, cols[j]], v)
        return carry

    lax.fori_loop(0, nch // 8, body, 0)
    pltpu.sync_copy(
        packed_v.at[pl.ds(0, ncams)],
        packed_out.at[pl.ds(cam0, ncams)])


def _pack_body(rott, trt, ppt, packed_out, colstack, packed_v, sem):
    wid = lax.axis_index("s") * _NUM_CORES + lax.axis_index("c")

    @pl.when(wid < _NW - 1)
    def _():
        _pack_block(rott, trt, ppt, packed_out, colstack, packed_v, sem,
                    wid * _CPT, _CPT, _NCH)

    @pl.when(wid == _NW - 1)
    def _():
        _pack_block(rott, trt, ppt, packed_out, colstack, packed_v, sem,
                    (_NW - 1) * _CPT, _CPT_LAST, _NCH_LAST)


_pack = functools.partial(
    pl.kernel,
    mesh=_mesh,
    compiler_params=_params,
    out_type=jax.ShapeDtypeStruct((_N, _W), jnp.float32),
    scratch_types=[
        pltpu.VMEM((_W, _PACK_PAD), jnp.float32),
        pltpu.VMEM((_PACK_PAD, _W), jnp.float32),
        pltpu.SemaphoreType.DMA,
    ],
)(_pack_body)


def _gather_body(tab, fv, idx_hbm, rvt_out, trt_out, fg_out, ppt_out,
                 idx_v, rows_v, fbuf, colbuf, sem, fsem):
    wid = lax.axis_index("s") * _NUM_CORES + lax.axis_index("c")
    base = wid * _BPW
    pltpu.sync_copy(idx_hbm.at[pl.ds(base, _BPW)], idx_v)
    copies = []
    for j in range(_BPW // _CHUNK):
        d = pl.ds(j * _CHUNK, _CHUNK)
        copies.append(pltpu.async_copy(tab.at[idx_v.at[d]], rows_v.at[d], sem))
        copies.append(pltpu.async_copy(fv.at[idx_v.at[d]], fbuf.at[d], fsem))
    for c in copies:
        c.wait()

    lane = lax.iota(jnp.int32, _L)
    for j in range(_W):
        jv = jnp.full((_L,), j, jnp.int32)
        for c in range(_BPW // _L):
            v = plsc.load_gather(rows_v, [c * _L + lane, jv])
            colbuf[j, pl.ds(c * _L, _L)] = v

    for j in range(3):
        pltpu.sync_copy(colbuf.at[j],
                        rvt_out.at[pl.ds(j * _BATCH + base, _BPW)])
        pltpu.sync_copy(colbuf.at[3 + j],
                        trt_out.at[pl.ds(j * _BATCH + base, _BPW)])
    for j in range(2):
        pltpu.sync_copy(colbuf.at[6 + j],
                        ppt_out.at[pl.ds(j * _BATCH + base, _BPW)])
    pltpu.sync_copy(fbuf, fg_out.at[pl.ds(base, _BPW)])


_gather = functools.partial(
    pl.kernel,
    mesh=_mesh,
    compiler_params=_params,
    out_type=(
        jax.ShapeDtypeStruct((3 * _BATCH,), jnp.float32),
        jax.ShapeDtypeStruct((3 * _BATCH,), jnp.float32),
        jax.ShapeDtypeStruct((_BATCH,), jnp.float32),
        jax.ShapeDtypeStruct((2 * _BATCH,), jnp.float32),
    ),
    scratch_types=[
        pltpu.VMEM((_BPW,), jnp.int32),
        pltpu.VMEM((_BPW, _W), jnp.float32),
        pltpu.VMEM((_BPW,), jnp.float32),
        pltpu.VMEM((_W, _BPW), jnp.float32),
        pltpu.SemaphoreType.DMA,
        pltpu.SemaphoreType.DMA,
    ],
)(_gather_body)


def kernel(rotvecs, translations, f, pp, camera_idxs):
    idx = camera_idxs.astype(jnp.int32)
    packed = _pack(rotvecs.T, translations.T, pp.T)
    rvt, trt, fg, ppt = _gather(packed, f, idx)
    return (rvt.reshape(3, _BATCH).T, trt.reshape(3, _BATCH).T, fg,
            ppt.reshape(2, _BATCH).T)


# speedup vs baseline: 1.0628x; 1.0021x over previous
import functools

import jax
import jax.numpy as jnp
from jax import lax
from jax.experimental import pallas as pl
from jax.experimental.pallas import tpu as pltpu
from jax.experimental.pallas import tpu_sc as plsc

_N = 100000
_BATCH = 16384
_NUM_CORES = 2
_NUM_SUBCORES = 16
_NW = _NUM_CORES * _NUM_SUBCORES
_BPW = _BATCH // _NW
_CHUNK = 512
_W = 8
_CPT = 3128
_CPT_LAST = _N - 31 * _CPT
_NCH = 200
_NCH_LAST = 192
_PACK_PAD = 3200
_L = 16

_mesh = plsc.VectorSubcoreMesh(
    core_axis_name="c",
    subcore_axis_name="s",
    num_cores=_NUM_CORES,
    num_subcores=_NUM_SUBCORES,
)
_params = pltpu.CompilerParams(
    use_tc_tiling_on_sc=False, needs_layout_passes=False)


def _pack_block(rott, trt, ppt, packed_out, colstack, packed_v, sem,
                cam0, ncams, nch):
    copies = []
    for j in range(3):
        copies.append(pltpu.async_copy(
            rott.at[j, pl.ds(cam0, ncams)], colstack.at[j, pl.ds(0, ncams)],
            sem))
        copies.append(pltpu.async_copy(
            trt.at[j, pl.ds(cam0, ncams)],
            colstack.at[3 + j, pl.ds(0, ncams)], sem))
    for j in range(2):
        copies.append(pltpu.async_copy(
            ppt.at[j, pl.ds(cam0, ncams)],
            colstack.at[6 + j, pl.ds(0, ncams)], sem))
    for c in copies:
        c.wait()

    lane = lax.iota(jnp.int32, _L)
    cols = [jnp.full((_L,), j, jnp.int32) for j in range(_W)]

    def body(c, carry):
        for k in range(8):
            ch = c * 8 + k
            rows = lane + ch * _L
            for j in range(_W):
                v = colstack[j, pl.ds(ch * _L, _L)]
                plsc.store_scatter(packed_v, [rows, cols[j]], v)
        return carry

    lax.fori_loop(0, nch // 8, body, 0)
    pltpu.sync_copy(
        packed_v.at[pl.ds(0, ncams)],
        packed_out.at[pl.ds(cam0, ncams)])


def _pack_body(rott, trt, ppt, packed_out, colstack, packed_v, sem):
    wid = lax.axis_index("s") * _NUM_CORES + lax.axis_index("c")

    @pl.when(wid < _NW - 1)
    def _():
        _pack_block(rott, trt, ppt, packed_out, colstack, packed_v, sem,
                    wid * _CPT, _CPT, _NCH)

    @pl.when(wid == _NW - 1)
    def _():
        _pack_block(rott, trt, ppt, packed_out, colstack, packed_v, sem,
                    (_NW - 1) * _CPT, _CPT_LAST, _NCH_LAST)


_pack = functools.partial(
    pl.kernel,
    mesh=_mesh,
    compiler_params=_params,
    out_type=jax.ShapeDtypeStruct((_N, _W), jnp.float32),
    scratch_types=[
        pltpu.VMEM((_W, _PACK_PAD), jnp.float32),
        pltpu.VMEM((_PACK_PAD, _W), jnp.float32),
        pltpu.SemaphoreType.DMA,
    ],
)(_pack_body)


def _gather_body(tab, fv, idx_hbm, rvt_out, trt_out, fg_out, ppt_out,
                 idx_v, rows_v, fbuf, colbuf, sem, fsem):
    wid = lax.axis_index("s") * _NUM_CORES + lax.axis_index("c")
    base = wid * _BPW
    pltpu.sync_copy(idx_hbm.at[pl.ds(base, _BPW)], idx_v)
    copies = []
    for j in range(_BPW // _CHUNK):
        d = pl.ds(j * _CHUNK, _CHUNK)
        copies.append(pltpu.async_copy(tab.at[idx_v.at[d]], rows_v.at[d], sem))
        copies.append(pltpu.async_copy(fv.at[idx_v.at[d]], fbuf.at[d], fsem))
    for c in copies:
        c.wait()

    lane = lax.iota(jnp.int32, _L)
    for j in range(_W):
        jv = jnp.full((_L,), j, jnp.int32)
        for c in range(_BPW // _L):
            v = plsc.load_gather(rows_v, [c * _L + lane, jv])
            colbuf[j, pl.ds(c * _L, _L)] = v

    for j in range(3):
        pltpu.sync_copy(colbuf.at[j],
                        rvt_out.at[pl.ds(j * _BATCH + base, _BPW)])
        pltpu.sync_copy(colbuf.at[3 + j],
                        trt_out.at[pl.ds(j * _BATCH + base, _BPW)])
    for j in range(2):
        pltpu.sync_copy(colbuf.at[6 + j],
                        ppt_out.at[pl.ds(j * _BATCH + base, _BPW)])
    pltpu.sync_copy(fbuf, fg_out.at[pl.ds(base, _BPW)])


_gather = functools.partial(
    pl.kernel,
    mesh=_mesh,
    compiler_params=_params,
    out_type=(
        jax.ShapeDtypeStruct((3 * _BATCH,), jnp.float32),
        jax.ShapeDtypeStruct((3 * _BATCH,), jnp.float32),
        jax.ShapeDtypeStruct((_BATCH,), jnp.float32),
        jax.ShapeDtypeStruct((2 * _BATCH,), jnp.float32),
    ),
    scratch_types=[
        pltpu.VMEM((_BPW,), jnp.int32),
        pltpu.VMEM((_BPW, _W), jnp.float32),
        pltpu.VMEM((_BPW,), jnp.float32),
        pltpu.VMEM((_W, _BPW), jnp.float32),
        pltpu.SemaphoreType.DMA,
        pltpu.SemaphoreType.DMA,
    ],
)(_gather_body)


def kernel(rotvecs, translations, f, pp, camera_idxs):
    idx = camera_idxs.astype(jnp.int32)
    packed = _pack(rotvecs.T, translations.T, pp.T)
    rvt, trt, fg, ppt = _gather(packed, f, idx)
    return (rvt.reshape(3, _BATCH).T, trt.reshape(3, _BATCH).T, fg,
            ppt.reshape(2, _BATCH).T)


# speedup vs baseline: 1.1552x; 1.0869x over previous
import functools

import jax
import jax.numpy as jnp
from jax import lax
from jax.experimental import pallas as pl
from jax.experimental.pallas import tpu as pltpu
from jax.experimental.pallas import tpu_sc as plsc

_N = 100000
_BATCH = 16384
_NUM_CORES = 2
_NUM_SUBCORES = 16
_NW = _NUM_CORES * _NUM_SUBCORES
_BPW = _BATCH // _NW
_CHUNK = 512
_W = 8
_CPT = 3128
_CPT_LAST = _N - 31 * _CPT
_NCH = 200
_NCH_LAST = 192
_PACK_PAD = 3200
_L = 16

_mesh = plsc.VectorSubcoreMesh(
    core_axis_name="c",
    subcore_axis_name="s",
    num_cores=_NUM_CORES,
    num_subcores=_NUM_SUBCORES,
)
_params = pltpu.CompilerParams(
    use_tc_tiling_on_sc=False, needs_layout_passes=False)


def _pack_block(rott, trt, ppt, packed_out, colstack, packed_v, sem,
                cam0, ncams, nch):
    copies = []
    for j in range(3):
        copies.append(pltpu.async_copy(
            rott.at[j, pl.ds(cam0, ncams)], colstack.at[j, pl.ds(0, ncams)],
            sem))
        copies.append(pltpu.async_copy(
            trt.at[j, pl.ds(cam0, ncams)],
            colstack.at[3 + j, pl.ds(0, ncams)], sem))
    for j in range(2):
        copies.append(pltpu.async_copy(
            ppt.at[j, pl.ds(cam0, ncams)],
            colstack.at[6 + j, pl.ds(0, ncams)], sem))
    for c in copies:
        c.wait()

    lane = lax.iota(jnp.int32, _L)
    cols = [jnp.full((_L,), j, jnp.int32) for j in range(_W)]

    @plsc.parallel_loop(0, nch, unroll=8)
    def _(ch):
        rows = lane + ch * _L
        for j in range(_W):
            v = colstack[j, pl.ds(ch * _L, _L)]
            plsc.store_scatter(packed_v, [rows, cols[j]], v)
    pltpu.sync_copy(
        packed_v.at[pl.ds(0, ncams)],
        packed_out.at[pl.ds(cam0, ncams)])


def _pack_body(rott, trt, ppt, packed_out, colstack, packed_v, sem):
    wid = lax.axis_index("s") * _NUM_CORES + lax.axis_index("c")

    @pl.when(wid < _NW - 1)
    def _():
        _pack_block(rott, trt, ppt, packed_out, colstack, packed_v, sem,
                    wid * _CPT, _CPT, _NCH)

    @pl.when(wid == _NW - 1)
    def _():
        _pack_block(rott, trt, ppt, packed_out, colstack, packed_v, sem,
                    (_NW - 1) * _CPT, _CPT_LAST, _NCH_LAST)


_pack = functools.partial(
    pl.kernel,
    mesh=_mesh,
    compiler_params=_params,
    out_type=jax.ShapeDtypeStruct((_N, _W), jnp.float32),
    scratch_types=[
        pltpu.VMEM((_W, _PACK_PAD), jnp.float32),
        pltpu.VMEM((_PACK_PAD, _W), jnp.float32),
        pltpu.SemaphoreType.DMA,
    ],
)(_pack_body)


def _gather_body(tab, fv, idx_hbm, rvt_out, trt_out, fg_out, ppt_out,
                 idx_v, rows_v, fbuf, colbuf, sem, fsem):
    wid = lax.axis_index("s") * _NUM_CORES + lax.axis_index("c")
    base = wid * _BPW
    pltpu.sync_copy(idx_hbm.at[pl.ds(base, _BPW)], idx_v)
    copies = []
    for j in range(_BPW // _CHUNK):
        d = pl.ds(j * _CHUNK, _CHUNK)
        copies.append(pltpu.async_copy(tab.at[idx_v.at[d]], rows_v.at[d], sem))
        copies.append(pltpu.async_copy(fv.at[idx_v.at[d]], fbuf.at[d], fsem))
    for c in copies:
        c.wait()

    lane = lax.iota(jnp.int32, _L)
    for j in range(_W):
        jv = jnp.full((_L,), j, jnp.int32)
        for c in range(_BPW // _L):
            v = plsc.load_gather(rows_v, [c * _L + lane, jv])
            colbuf[j, pl.ds(c * _L, _L)] = v

    for j in range(3):
        pltpu.sync_copy(colbuf.at[j],
                        rvt_out.at[pl.ds(j * _BATCH + base, _BPW)])
        pltpu.sync_copy(colbuf.at[3 + j],
                        trt_out.at[pl.ds(j * _BATCH + base, _BPW)])
    for j in range(2):
        pltpu.sync_copy(colbuf.at[6 + j],
                        ppt_out.at[pl.ds(j * _BATCH + base, _BPW)])
    pltpu.sync_copy(fbuf, fg_out.at[pl.ds(base, _BPW)])


_gather = functools.partial(
    pl.kernel,
    mesh=_mesh,
    compiler_params=_params,
    out_type=(
        jax.ShapeDtypeStruct((3 * _BATCH,), jnp.float32),
        jax.ShapeDtypeStruct((3 * _BATCH,), jnp.float32),
        jax.ShapeDtypeStruct((_BATCH,), jnp.float32),
        jax.ShapeDtypeStruct((2 * _BATCH,), jnp.float32),
    ),
    scratch_types=[
        pltpu.VMEM((_BPW,), jnp.int32),
        pltpu.VMEM((_BPW, _W), jnp.float32),
        pltpu.VMEM((_BPW,), jnp.float32),
        pltpu.VMEM((_W, _BPW), jnp.float32),
        pltpu.SemaphoreType.DMA,
        pltpu.SemaphoreType.DMA,
    ],
)(_gather_body)


def kernel(rotvecs, translations, f, pp, camera_idxs):
    idx = camera_idxs.astype(jnp.int32)
    packed = _pack(rotvecs.T, translations.T, pp.T)
    rvt, trt, fg, ppt = _gather(packed, f, idx)
    return (rvt.reshape(3, _BATCH).T, trt.reshape(3, _BATCH).T, fg,
            ppt.reshape(2, _BATCH).T)


# speedup vs baseline: 1.2196x; 1.0557x over previous
import functools

import jax
import jax.numpy as jnp
from jax import lax
from jax.experimental import pallas as pl
from jax.experimental.pallas import tpu as pltpu
from jax.experimental.pallas import tpu_sc as plsc

_N = 100000
_BATCH = 16384
_NUM_CORES = 2
_NUM_SUBCORES = 16
_NW = _NUM_CORES * _NUM_SUBCORES
_BPW = _BATCH // _NW
_CHUNK = 512
_W = 8
_CPT = 3128
_CPT_LAST = _N - 31 * _CPT
_NCH = 200
_NCH_LAST = 192
_PACK_PAD = 3200
_L = 16

_mesh = plsc.VectorSubcoreMesh(
    core_axis_name="c",
    subcore_axis_name="s",
    num_cores=_NUM_CORES,
    num_subcores=_NUM_SUBCORES,
)
_params = pltpu.CompilerParams(
    use_tc_tiling_on_sc=False, needs_layout_passes=False)


def _pack_block(rott, trt, ppt, packed_out, colstack, packed_v, sem,
                cam0, ncams, nch):
    copies = []
    for j in range(3):
        copies.append(pltpu.async_copy(
            rott.at[j, pl.ds(cam0, ncams)], colstack.at[j, pl.ds(0, ncams)],
            sem))
        copies.append(pltpu.async_copy(
            trt.at[j, pl.ds(cam0, ncams)],
            colstack.at[3 + j, pl.ds(0, ncams)], sem))
    for j in range(2):
        copies.append(pltpu.async_copy(
            ppt.at[j, pl.ds(cam0, ncams)],
            colstack.at[6 + j, pl.ds(0, ncams)], sem))
    for c in copies:
        c.wait()

    lane = lax.iota(jnp.int32, _L)
    cols = [jnp.full((_L,), j, jnp.int32) for j in range(_W)]

    @plsc.parallel_loop(0, nch, unroll=8)
    def _(ch):
        rows = lane + ch * _L
        for j in range(_W):
            v = colstack[j, pl.ds(ch * _L, _L)]
            plsc.store_scatter(packed_v, [rows, cols[j]], v)
    pltpu.sync_copy(
        packed_v.at[pl.ds(0, ncams)],
        packed_out.at[pl.ds(cam0, ncams)])


def _pack_body(rott, trt, ppt, packed_out, colstack, packed_v, sem):
    wid = lax.axis_index("s") * _NUM_CORES + lax.axis_index("c")

    @pl.when(wid < _NW - 1)
    def _():
        _pack_block(rott, trt, ppt, packed_out, colstack, packed_v, sem,
                    wid * _CPT, _CPT, _NCH)

    @pl.when(wid == _NW - 1)
    def _():
        _pack_block(rott, trt, ppt, packed_out, colstack, packed_v, sem,
                    (_NW - 1) * _CPT, _CPT_LAST, _NCH_LAST)


_pack = functools.partial(
    pl.kernel,
    mesh=_mesh,
    compiler_params=_params,
    out_type=jax.ShapeDtypeStruct((_N, _W), jnp.float32),
    scratch_types=[
        pltpu.VMEM((_W, _PACK_PAD), jnp.float32),
        pltpu.VMEM((_PACK_PAD, _W), jnp.float32),
        pltpu.SemaphoreType.DMA,
    ],
)(_pack_body)


def _gather_body(tab, fv, idx_hbm, rvt_out, trt_out, fg_out, ppt_out,
                 idx_v, rows_v, fbuf, colbuf, sem, fsem):
    wid = lax.axis_index("s") * _NUM_CORES + lax.axis_index("c")
    base = wid * _BPW
    pltpu.sync_copy(idx_hbm.at[pl.ds(base, _BPW)], idx_v)
    copies = []
    for j in range(_BPW // _CHUNK):
        d = pl.ds(j * _CHUNK, _CHUNK)
        copies.append(pltpu.async_copy(tab.at[idx_v.at[d]], rows_v.at[d], sem))
        copies.append(pltpu.async_copy(fv.at[idx_v.at[d]], fbuf.at[d], fsem))
    for c in copies:
        c.wait()

    lane = lax.iota(jnp.int32, _L)
    jvs = [jnp.full((_L,), j, jnp.int32) for j in range(_W)]

    @plsc.parallel_loop(0, _BPW // _L, unroll=8)
    def _(c):
        rows = c * _L + lane
        for j in range(_W):
            v = plsc.load_gather(rows_v, [rows, jvs[j]])
            colbuf[j, pl.ds(c * _L, _L)] = v

    for j in range(3):
        pltpu.sync_copy(colbuf.at[j],
                        rvt_out.at[pl.ds(j * _BATCH + base, _BPW)])
        pltpu.sync_copy(colbuf.at[3 + j],
                        trt_out.at[pl.ds(j * _BATCH + base, _BPW)])
    for j in range(2):
        pltpu.sync_copy(colbuf.at[6 + j],
                        ppt_out.at[pl.ds(j * _BATCH + base, _BPW)])
    pltpu.sync_copy(fbuf, fg_out.at[pl.ds(base, _BPW)])


_gather = functools.partial(
    pl.kernel,
    mesh=_mesh,
    compiler_params=_params,
    out_type=(
        jax.ShapeDtypeStruct((3 * _BATCH,), jnp.float32),
        jax.ShapeDtypeStruct((3 * _BATCH,), jnp.float32),
        jax.ShapeDtypeStruct((_BATCH,), jnp.float32),
        jax.ShapeDtypeStruct((2 * _BATCH,), jnp.float32),
    ),
    scratch_types=[
        pltpu.VMEM((_BPW,), jnp.int32),
        pltpu.VMEM((_BPW, _W), jnp.float32),
        pltpu.VMEM((_BPW,), jnp.float32),
        pltpu.VMEM((_W, _BPW), jnp.float32),
        pltpu.SemaphoreType.DMA,
        pltpu.SemaphoreType.DMA,
    ],
)(_gather_body)


def kernel(rotvecs, translations, f, pp, camera_idxs):
    idx = camera_idxs.astype(jnp.int32)
    packed = _pack(rotvecs.T, translations.T, pp.T)
    rvt, trt, fg, ppt = _gather(packed, f, idx)
    return (rvt.reshape(3, _BATCH).T, trt.reshape(3, _BATCH).T, fg,
            ppt.reshape(2, _BATCH).T)
